# baseline (speedup 1.0000x reference)
import functools

import jax
import jax.numpy as jnp
from jax import lax
from jax.experimental import pallas as pl
from jax.experimental.pallas import tpu as pltpu
from jax.experimental.pallas import tpu_sc as plsc

_NC = 2
_NS = 16
_NW = _NC * _NS
_TH = 4
_BW = 128
_PITCH = 129


def kernel(X, word_table, pos_table):
    B, T = X.shape
    V, D = word_table.shape
    EB, E8, BB = D // 8, 8, B // _BW
    iters = T // (2 * _TH)

    xt = X.T

    mesh = plsc.VectorSubcoreMesh(core_axis_name="c", subcore_axis_name="s")

    @functools.partial(
        pl.kernel,
        out_type=jax.ShapeDtypeStruct((T, EB, BB, E8, _BW), jnp.float32),
        mesh=mesh,
        scratch_types=[
            pltpu.VMEM((T, _BW), jnp.int32),
            pltpu.VMEM((_TH * _BW, D), jnp.float32),
            pltpu.VMEM((_TH * _BW, D), jnp.float32),
            pltpu.VMEM((D, _PITCH), jnp.float32),
            pltpu.VMEM((T, D), jnp.float32),
            pltpu.SemaphoreType.DMA,
            pltpu.SemaphoreType.DMA,
        ],
        compiler_params=pltpu.CompilerParams(
            use_tc_tiling_on_sc=False, needs_layout_passes=False
        ),
    )
    def emb(
        xt_hbm, tab_hbm, pos_hbm, p_hbm,
        idx_v, rows_a, rows_b, blk_v, pos_v, sem_a, sem_b,
    ):
        wid = lax.axis_index("s") * _NC + lax.axis_index("c")
        b0 = wid * _BW
        pltpu.sync_copy(pos_hbm, pos_v)
        pltpu.sync_copy(xt_hbm.at[pl.ds(0, T), pl.ds(b0, _BW)], idx_v)
        lane = lax.iota(jnp.int32, 16)
        evecs = [lane + 16 * c for c in range(D // 16)]
        nc = D // 16

        def fire(buf, sem, tbase):
            return [
                pltpu.async_copy(
                    tab_hbm.at[idx_v.at[jnp.minimum(tbase + j, T - 1)]],
                    buf.at[pl.ds(j * _BW, _BW)],
                    sem,
                )
                for j in range(_TH)
            ]

        def drain(buf, sem):
            for j in range(_TH):
                pltpu.make_async_copy(
                    tab_hbm.at[pl.ds(0, _BW)],
                    buf.at[pl.ds(j * _BW, _BW)],
                    sem,
                ).wait()

        def process(buf, tbase):
            def t_body(j, c2):
                t = tbase + j
                pvs = [pos_v[t, pl.ds(16 * c, 16)] for c in range(nc)]
                for rp in range(_BW // 2):
                    rows = (j * _BW + 2 * rp, j * _BW + 2 * rp + 1)
                    bvecs = (
                        jnp.full((16,), 2 * rp, dtype=jnp.int32),
                        jnp.full((16,), 2 * rp + 1, dtype=jnp.int32),
                    )
                    ws = [
                        buf[rows[q], pl.ds(16 * c, 16)] + pvs[c]
                        for q in range(2)
                        for c in range(nc)
                    ]
                    for q in range(2):
                        for c in range(nc):
                            plsc.store_scatter(
                                blk_v,
                                [evecs[c], bvecs[q]],
                                ws[q * nc + c],
                            )
                for eb in range(EB):
                    pltpu.sync_copy(
                        blk_v.at[pl.ds(eb * E8, E8), pl.ds(0, _BW)],
                        p_hbm.at[t, eb, wid],
                    )
                return c2

            lax.fori_loop(0, _TH, t_body, 0)

        fire(rows_a, sem_a, 0)

        def body(it, carry):
            t0 = it * 2 * _TH
            cps_b = fire(rows_b, sem_b, t0 + _TH)
            drain(rows_a, sem_a)
            process(rows_a, t0)
            fire(rows_a, sem_a, t0 + 2 * _TH)
            for cp in cps_b:
                cp.wait()
            process(rows_b, t0 + _TH)
            return carry

        lax.fori_loop(0, iters, body, 0)
        drain(rows_a, sem_a)

    p = emb(xt, word_table, pos_table)
    return p.transpose((2, 4, 0, 1, 3)).reshape(B, T, D)


# speedup vs baseline: 1.1994x; 1.1994x over previous
import functools

import jax
import jax.numpy as jnp
from jax import lax
from jax.experimental import pallas as pl
from jax.experimental.pallas import tpu as pltpu
from jax.experimental.pallas import tpu_sc as plsc

_NC = 2
_NS = 16
_NW = _NC * _NS
_TC = 8
_BW = 128
_PITCH = 129


def kernel(X, word_table, pos_table):
    B, T = X.shape
    V, D = word_table.shape
    EB, E8, BB = D // 8, 8, B // _BW
    chunks = T // _TC

    xt = X.T

    mesh = plsc.VectorSubcoreMesh(core_axis_name="c", subcore_axis_name="s")

    @functools.partial(
        pl.kernel,
        out_type=jax.ShapeDtypeStruct((T, EB, BB, E8, _BW), jnp.float32),
        mesh=mesh,
        scratch_types=[
            pltpu.VMEM((_TC, _BW), jnp.int32),
            pltpu.VMEM((_TC * _BW, D), jnp.float32),
            pltpu.VMEM((D, _PITCH), jnp.float32),
            pltpu.VMEM((T, D), jnp.float32),
            pltpu.SemaphoreType.DMA((_TC,)),
        ],
        compiler_params=pltpu.CompilerParams(
            use_tc_tiling_on_sc=False, needs_layout_passes=False
        ),
    )
    def emb(xt_hbm, tab_hbm, pos_hbm, p_hbm, idx_v, rows_v, blk_v, pos_v, sem):
        wid = lax.axis_index("s") * _NC + lax.axis_index("c")
        b0 = wid * _BW
        pltpu.sync_copy(pos_hbm, pos_v)
        lane = lax.iota(jnp.int32, 16)
        evecs = [lane + 16 * c for c in range(D // 16)]

        def chunk_body(it, carry):
            t0 = pl.multiple_of(it * _TC, _TC)
            pltpu.sync_copy(
                xt_hbm.at[pl.ds(t0, _TC), pl.ds(b0, _BW)], idx_v
            )
            for ti in range(_TC):
                pltpu.async_copy(
                    tab_hbm.at[idx_v.at[ti]],
                    rows_v.at[pl.ds(ti * _BW, _BW)],
                    sem.at[ti],
                )

            def t_body(ti, c2):
                t = t0 + ti
                pltpu.make_async_copy(
                    tab_hbm.at[pl.ds(0, _BW)],
                    rows_v.at[pl.ds(0, _BW)],
                    sem.at[ti],
                ).wait()
                pvs = [pos_v[t, pl.ds(16 * c, 16)] for c in range(D // 16)]
                nc = D // 16
                for rp in range(_BW // 2):
                    rows = (ti * _BW + 2 * rp, ti * _BW + 2 * rp + 1)
                    bvecs = (
                        jnp.full((16,), 2 * rp, dtype=jnp.int32),
                        jnp.full((16,), 2 * rp + 1, dtype=jnp.int32),
                    )
                    ws = [
                        rows_v[rows[j], pl.ds(16 * c, 16)] + pvs[c]
                        for j in range(2)
                        for c in range(nc)
                    ]
                    for j in range(2):
                        for c in range(nc):
                            plsc.store_scatter(
                                blk_v,
                                [evecs[c], bvecs[j]],
                                ws[j * nc + c],
                            )
                for eb in range(EB):
                    pltpu.sync_copy(
                        blk_v.at[pl.ds(eb * E8, E8), pl.ds(0, _BW)],
                        p_hbm.at[t, eb, wid],
                    )
                return c2

            lax.fori_loop(0, _TC, t_body, 0)
            return carry

        lax.fori_loop(0, chunks, chunk_body, 0)

    p = emb(xt, word_table, pos_table)
    return p.transpose((2, 4, 0, 1, 3)).reshape(B, T, D)
